# Initial kernel scaffold; baseline (speedup 1.0000x reference)
import jax
import jax.numpy as jnp
from jax.experimental import pallas as pl


def kernel(x, cliques_r, cliques_s, node_features, lin5_w, lin5_b, lin6_w, lin6_b):
    raise NotImplementedError("write your pallas kernel here")


# speedup vs baseline: 2.4887x; 2.4887x over previous
import functools

import jax
import jax.numpy as jnp
from jax import lax
from jax.experimental import pallas as pl
from jax.experimental.pallas import tpu as pltpu
from jax.experimental.pallas import tpu_sc as plsc

F = 128
N_PER_SET = 50000
NC = 2
NS = 16
NW = NC * NS
HALF = 102400
PAD = HALF - 2 * N_PER_SET
B_ALL = 2 * HALF
PER_W = B_ALL // NW
NB = 128
NBATCH = PER_W // NB

BLK = 512


def _sc_gather_body(table_hbm, idx_hbm, out_hbm, idx_v, buf, sem):
  wid = lax.axis_index("s") * NC + lax.axis_index("c")
  base = wid * PER_W
  pltpu.sync_copy(idx_hbm.at[pl.ds(base, PER_W)], idx_v)

  def step(t, carry):
    cp = pltpu.make_async_copy(
        table_hbm.at[idx_v.at[pl.ds(t * NB, NB)]], buf, sem)
    cp.start()
    cp.wait()
    pltpu.sync_copy(buf, out_hbm.at[pl.ds(base + t * NB, NB)])
    return carry

  lax.fori_loop(0, NBATCH, step, 0)


@functools.lru_cache(maxsize=None)
def _make_sc_gather():
  return pl.kernel(
      _sc_gather_body,
      out_type=jax.ShapeDtypeStruct((B_ALL, F), jnp.float32),
      mesh=plsc.VectorSubcoreMesh(core_axis_name="c", subcore_axis_name="s"),
      scratch_types=[
          pltpu.VMEM((PER_W,), jnp.int32),
          pltpu.VMEM((NB, F), jnp.float32),
          pltpu.SemaphoreType.DMA,
      ],
  )


def _mlp_body(xi_ref, xj_ref, w5_ref, b5_ref, w6_ref, b6_ref, out_ref):
  p = xi_ref[...] * xj_ref[...]
  h = lax.dot_general(
      p, w5_ref[...], (((1,), (1,)), ((), ())),
      preferred_element_type=jnp.float32,
      precision=lax.Precision.HIGHEST)
  h = jnp.maximum(h + b5_ref[...], 0.0)
  logits = lax.dot_general(
      h, w6_ref[...], (((1,), (1,)), ((), ())),
      preferred_element_type=jnp.float32,
      precision=lax.Precision.HIGHEST) + b6_ref[...]
  m = jnp.max(logits, axis=-1, keepdims=True)
  e = jnp.exp(logits - m)
  out_ref[...] = e / jnp.sum(e, axis=-1, keepdims=True)


def kernel(x, cliques_r, cliques_s, node_features, lin5_w, lin5_b, lin6_w,
           lin6_b):
  del x
  zpad = jnp.zeros((PAD,), jnp.int32)
  idx_all = jnp.concatenate([
      cliques_r[:, 0], cliques_s[:, 0], zpad,
      cliques_r[:, 1], cliques_s[:, 1], zpad,
  ])
  gathered = _make_sc_gather()(node_features, idx_all)

  noff = HALF // BLK
  probs = pl.pallas_call(
      _mlp_body,
      grid=(noff,),
      in_specs=[
          pl.BlockSpec((BLK, F), lambda i: (i, 0)),
          pl.BlockSpec((BLK, F), lambda i: (i + noff, 0)),
          pl.BlockSpec((F, F), lambda i: (0, 0)),
          pl.BlockSpec((1, F), lambda i: (0, 0)),
          pl.BlockSpec((2, F), lambda i: (0, 0)),
          pl.BlockSpec((1, 2), lambda i: (0, 0)),
      ],
      out_specs=pl.BlockSpec((BLK, 2), lambda i: (i, 0)),
      out_shape=jax.ShapeDtypeStruct((HALF, 2), jnp.float32),
  )(gathered, gathered, lin5_w, lin5_b.reshape(1, F), lin6_w,
    lin6_b.reshape(1, 2))

  edge_probs_r = probs[:N_PER_SET]
  edge_probs_s = probs[N_PER_SET:2 * N_PER_SET][:, None, :]
  return (edge_probs_r, edge_probs_s)


# speedup vs baseline: 2.6518x; 1.0656x over previous
import functools

import jax
import jax.numpy as jnp
from jax import lax
from jax.experimental import pallas as pl
from jax.experimental.pallas import tpu as pltpu
from jax.experimental.pallas import tpu_sc as plsc

F = 128
N_PER_SET = 50000
NC = 2
NS = 16
NW = NC * NS
HALF = 102400
PAD = HALF - 2 * N_PER_SET
B_ALL = 2 * HALF
PER_W = B_ALL // NW
NB = 128
NBATCH = PER_W // NB

BLK = 512


def _sc_gather_body(table_hbm, idx_hbm, out_hbm, idx_v, buf0, buf1, gsem0,
                    gsem1, wsem0, wsem1):
  wid = lax.axis_index("s") * NC + lax.axis_index("c")
  base = wid * PER_W
  pltpu.sync_copy(idx_hbm.at[pl.ds(base, PER_W)], idx_v)

  def gath(t, buf, sem):
    return pltpu.make_async_copy(
        table_hbm.at[idx_v.at[pl.ds(t * NB, NB)]], buf, sem)

  def wout(t, buf, sem):
    return pltpu.make_async_copy(buf, out_hbm.at[pl.ds(base + t * NB, NB)],
                                 sem)

  gath(0, buf0, gsem0).start()

  def step(u, carry):
    t0 = 2 * u
    gath(t0 + 1, buf1, gsem1).start()
    gath(t0, buf0, gsem0).wait()
    wout(t0, buf0, wsem0).start()
    wout(t0, buf0, wsem0).wait()

    @pl.when(u + 1 < NBATCH // 2)
    def _():
      gath(t0 + 2, buf0, gsem0).start()

    gath(t0 + 1, buf1, gsem1).wait()
    wout(t0 + 1, buf1, wsem1).start()
    wout(t0 + 1, buf1, wsem1).wait()
    return carry

  lax.fori_loop(0, NBATCH // 2, step, 0)


@functools.lru_cache(maxsize=None)
def _make_sc_gather():
  return pl.kernel(
      _sc_gather_body,
      out_type=jax.ShapeDtypeStruct((B_ALL, F), jnp.float32),
      mesh=plsc.VectorSubcoreMesh(core_axis_name="c", subcore_axis_name="s"),
      scratch_types=[
          pltpu.VMEM((PER_W,), jnp.int32),
          pltpu.VMEM((NB, F), jnp.float32),
          pltpu.VMEM((NB, F), jnp.float32),
          pltpu.SemaphoreType.DMA,
          pltpu.SemaphoreType.DMA,
          pltpu.SemaphoreType.DMA,
          pltpu.SemaphoreType.DMA,
      ],
  )


def _mlp_body(xi_ref, xj_ref, w5_ref, b5_ref, w6_ref, b6_ref, out_ref):
  p = xi_ref[...] * xj_ref[...]
  h = lax.dot_general(
      p, w5_ref[...], (((1,), (1,)), ((), ())),
      preferred_element_type=jnp.float32,
      precision=lax.Precision.HIGHEST)
  h = jnp.maximum(h + b5_ref[...], 0.0)
  logits = lax.dot_general(
      h, w6_ref[...], (((1,), (1,)), ((), ())),
      preferred_element_type=jnp.float32,
      precision=lax.Precision.HIGHEST) + b6_ref[...]
  m = jnp.max(logits, axis=-1, keepdims=True)
  e = jnp.exp(logits - m)
  out_ref[...] = e / jnp.sum(e, axis=-1, keepdims=True)


def kernel(x, cliques_r, cliques_s, node_features, lin5_w, lin5_b, lin6_w,
           lin6_b):
  del x
  zpad = jnp.zeros((PAD,), jnp.int32)
  idx_all = jnp.concatenate([
      cliques_r[:, 0], cliques_s[:, 0], zpad,
      cliques_r[:, 1], cliques_s[:, 1], zpad,
  ])
  gathered = _make_sc_gather()(node_features, idx_all)

  noff = HALF // BLK
  probs = pl.pallas_call(
      _mlp_body,
      grid=(noff,),
      in_specs=[
          pl.BlockSpec((BLK, F), lambda i: (i, 0)),
          pl.BlockSpec((BLK, F), lambda i: (i + noff, 0)),
          pl.BlockSpec((F, F), lambda i: (0, 0)),
          pl.BlockSpec((1, F), lambda i: (0, 0)),
          pl.BlockSpec((2, F), lambda i: (0, 0)),
          pl.BlockSpec((1, 2), lambda i: (0, 0)),
      ],
      out_specs=pl.BlockSpec((BLK, 2), lambda i: (i, 0)),
      out_shape=jax.ShapeDtypeStruct((HALF, 2), jnp.float32),
  )(gathered, gathered, lin5_w, lin5_b.reshape(1, F), lin6_w,
    lin6_b.reshape(1, 2))

  edge_probs_r = probs[:N_PER_SET]
  edge_probs_s = probs[N_PER_SET:2 * N_PER_SET][:, None, :]
  return (edge_probs_r, edge_probs_s)
